# baseline (speedup 1.0000x reference)
import functools

import jax
import jax.numpy as jnp
from jax import lax
from jax.experimental import pallas as pl
from jax.experimental.pallas import tpu as pltpu
from jax.experimental.pallas import tpu_sc as plsc

_VOCAB = 1000
_VPAD = 1024
_B, _L = 1024, 50
_N = _B * _L

_info = plsc.get_sparse_core_info()
_NC = _info.num_cores
_NS = _info.num_subcores
_NW = _NC * _NS
_ROWS_PW = _B // _NW
_LPAD = 56
_IPW = _ROWS_PW * _LPAD

_mesh = plsc.VectorSubcoreMesh(core_axis_name="c", subcore_axis_name="s")


@functools.partial(
    pl.kernel,
    mesh=_mesh,
    out_type=jax.ShapeDtypeStruct((_B, _LPAD, _VPAD), jnp.float32),
    scratch_types=[
        pltpu.VMEM((_IPW,), jnp.int32),
        [pltpu.VMEM((_LPAD, _VPAD), jnp.float32)] * 2,
        [pltpu.SemaphoreType.DMA] * 2,
        [pltpu.SemaphoreType.DMA] * 2,
    ],
)
def _gather_rows(idx_hbm, table_hbm, out_hbm, idx_v, bufs, semg, semw):
    wid = lax.axis_index("s") * _NC + lax.axis_index("c")
    base = wid * _IPW
    b0 = wid * _ROWS_PW
    pltpu.sync_copy(idx_hbm.at[pl.ds(base, _IPW)], idx_v)

    def gather(b, buf, sem):
        return pltpu.make_async_copy(
            table_hbm.at[idx_v.at[pl.ds(b * _LPAD, _LPAD)]], buf, sem)

    def write(b, buf, sem):
        return pltpu.make_async_copy(buf, out_hbm.at[b0 + b], sem)

    gather(0, bufs[0], semg[0]).start()

    def body(b, carry):
        for p in (0, 1):
            gather(b + p, bufs[p], semg[p]).wait()

            @pl.when(b + p >= 1)
            def _():
                write(b + p - 1, bufs[1 - p], semw[1 - p]).wait()

            @pl.when(b + p + 1 < _ROWS_PW)
            def _():
                gather(b + p + 1, bufs[1 - p], semg[1 - p]).start()

            write(b + p, bufs[p], semw[p]).start()
        return carry

    lax.fori_loop(0, _ROWS_PW // 2, lambda i, c: body(i * 2, c), 0)
    write(_ROWS_PW - 1, bufs[1], semw[1]).wait()


_BB = 4


def _slice_body(i_ref, o_ref):
    o_ref[...] = i_ref[:, : _L, : _VOCAB]


_slice_lanes = pl.pallas_call(
    _slice_body,
    grid=(_B // _BB,),
    in_specs=[pl.BlockSpec((_BB, _LPAD, _VPAD), lambda b: (b, 0, 0))],
    out_specs=pl.BlockSpec((_BB, _L, _VOCAB), lambda b: (b, 0, 0)),
    out_shape=jax.ShapeDtypeStruct((_B, _L, _VOCAB), jnp.float32),
)


def kernel(idx, logits):
    idx_p = jnp.pad(idx.astype(jnp.int32), ((0, 0), (0, _LPAD - _L)))
    flat = idx_p.reshape(_B * _LPAD)
    table = jnp.pad(logits, ((0, 0), (0, _VPAD - _VOCAB)))
    padded = _gather_rows(flat, table)
    return _slice_lanes(padded)


# speedup vs baseline: 1.5918x; 1.5918x over previous
import functools

import jax
import jax.numpy as jnp
from jax import lax
from jax.experimental import pallas as pl
from jax.experimental.pallas import tpu as pltpu
from jax.experimental.pallas import tpu_sc as plsc

_VOCAB = 1000
_B, _L = 1024, 50
_N = _B * _L

_info = plsc.get_sparse_core_info()
_NC = _info.num_cores
_NS = _info.num_subcores
_NW = _NC * _NS
_ROWS_PW = _B // _NW
_LPAD = 56
_IPW = _ROWS_PW * _LPAD

_mesh = plsc.VectorSubcoreMesh(core_axis_name="c", subcore_axis_name="s")


@functools.partial(
    pl.kernel,
    mesh=_mesh,
    out_type=jax.ShapeDtypeStruct((_B, _L, _VOCAB), jnp.float32),
    scratch_types=[
        pltpu.VMEM((_IPW,), jnp.int32),
        [pltpu.VMEM((_L, _VOCAB), jnp.float32)] * 2,
        [pltpu.SemaphoreType.DMA] * 2,
        [pltpu.SemaphoreType.DMA] * 2,
    ],
    compiler_params=pltpu.CompilerParams(use_tc_tiling_on_sc=False),
)
def _gather_rows(idx_hbm, table_hbm, out_hbm, idx_v, bufs, semg, semw):
    wid = lax.axis_index("s") * _NC + lax.axis_index("c")
    b0 = wid * _ROWS_PW
    pltpu.sync_copy(idx_hbm.at[pl.ds(wid * _IPW, _IPW)], idx_v)

    def gather(b, buf, sem):
        return pltpu.make_async_copy(
            table_hbm.at[idx_v.at[pl.ds(b * _LPAD, _L)]], buf, sem)

    def write(b, buf, sem):
        return pltpu.make_async_copy(buf, out_hbm.at[b0 + b], sem)

    gather(0, bufs[0], semg[0]).start()

    def body(b, carry):
        for p in (0, 1):
            gather(b + p, bufs[p], semg[p]).wait()

            @pl.when(b + p >= 1)
            def _():
                write(b + p - 1, bufs[1 - p], semw[1 - p]).wait()

            @pl.when(b + p + 1 < _ROWS_PW)
            def _():
                gather(b + p + 1, bufs[1 - p], semg[1 - p]).start()

            write(b + p, bufs[p], semw[p]).start()
        return carry

    lax.fori_loop(0, _ROWS_PW // 2, lambda i, c: body(i * 2, c), 0)
    write(_ROWS_PW - 1, bufs[1], semw[1]).wait()


def kernel(idx, logits):
    idx_p = jnp.pad(idx.astype(jnp.int32), ((0, 0), (0, _LPAD - _L)))
    return _gather_rows(idx_p.reshape(_B * _LPAD), logits)


# speedup vs baseline: 1.6982x; 1.0669x over previous
import functools

import jax
import jax.numpy as jnp
from jax import lax
from jax.experimental import pallas as pl
from jax.experimental.pallas import tpu as pltpu
from jax.experimental.pallas import tpu_sc as plsc

_VOCAB = 1000
_VPAD = 1024
_B, _L = 1024, 50
_N = _B * _L

_info = plsc.get_sparse_core_info()
_NC = _info.num_cores
_NS = _info.num_subcores
_NW = _NC * _NS
_ROWS_PW = _B // _NW
_LPAD = 56
_IPW = _ROWS_PW * _LPAD

_mesh = plsc.VectorSubcoreMesh(core_axis_name="c", subcore_axis_name="s")


@functools.partial(
    pl.kernel,
    mesh=_mesh,
    out_type=jax.ShapeDtypeStruct((_N, _VPAD), jnp.float32),
    scratch_types=[
        pltpu.VMEM((_IPW,), jnp.int32),
        [pltpu.VMEM((_L, _VPAD), jnp.float32)] * 2,
        [pltpu.SemaphoreType.DMA] * 2,
        [pltpu.SemaphoreType.DMA] * 2,
    ],
    compiler_params=pltpu.CompilerParams(use_tc_tiling_on_sc=False),
)
def _gather_rows(idx_hbm, table_hbm, out_hbm, idx_v, bufs, semg, semw):
    wid = lax.axis_index("s") * _NC + lax.axis_index("c")
    b0 = wid * _ROWS_PW
    pltpu.sync_copy(idx_hbm.at[pl.ds(wid * _IPW, _IPW)], idx_v)

    def gather(b, buf, sem):
        return pltpu.make_async_copy(
            table_hbm.at[idx_v.at[pl.ds(b * _LPAD, _L)]], buf, sem)

    def write(b, buf, sem):
        return pltpu.make_async_copy(
            buf, out_hbm.at[pl.ds((b0 + b) * _L, _L)], sem)

    gather(0, bufs[0], semg[0]).start()

    def body(b, carry):
        for p in (0, 1):
            gather(b + p, bufs[p], semg[p]).wait()

            @pl.when(b + p >= 1)
            def _():
                write(b + p - 1, bufs[1 - p], semw[1 - p]).wait()

            @pl.when(b + p + 1 < _ROWS_PW)
            def _():
                gather(b + p + 1, bufs[1 - p], semg[1 - p]).start()

            write(b + p, bufs[p], semw[p]).start()
        return carry

    lax.fori_loop(0, _ROWS_PW // 2, lambda i, c: body(i * 2, c), 0)
    write(_ROWS_PW - 1, bufs[1], semw[1]).wait()


_BB = 4


def _untile_body(x_ref, o_ref):
    x = x_ref[...]
    y = x.reshape(_BB * _L, _VPAD)
    y = y.reshape(_BB, _L, _VPAD)
    o_ref[...] = y[:, :, :_VOCAB]


_untile = pl.pallas_call(
    _untile_body,
    grid=(_B // _BB,),
    in_specs=[pl.BlockSpec((_BB * _L, 8, 128), lambda b: (b, 0, 0))],
    out_specs=pl.BlockSpec((_BB, _L, _VOCAB), lambda b: (b, 0, 0)),
    out_shape=jax.ShapeDtypeStruct((_B, _L, _VOCAB), jnp.float32),
)


def kernel(idx, logits):
    idx_p = jnp.pad(idx.astype(jnp.int32), ((0, 0), (0, _LPAD - _L)))
    table = jnp.pad(logits, ((0, 0), (0, _VPAD - _VOCAB)))
    rows = _gather_rows(idx_p.reshape(_B * _LPAD), table)
    return _untile(rows.reshape(_N, 8, 128))


# speedup vs baseline: 1.7001x; 1.0011x over previous
import functools

import jax
import jax.numpy as jnp
from jax import lax
from jax.experimental import pallas as pl
from jax.experimental.pallas import tpu as pltpu
from jax.experimental.pallas import tpu_sc as plsc

_VOCAB = 1000
_VPAD = 1024
_B, _L = 1024, 50
_N = _B * _L

_info = plsc.get_sparse_core_info()
_NC = _info.num_cores
_NS = _info.num_subcores
_NW = _NC * _NS
_ROWS_PW = _B // _NW
_LPAD = 56
_IPW = _ROWS_PW * _LPAD

_mesh = plsc.VectorSubcoreMesh(core_axis_name="c", subcore_axis_name="s")


@functools.partial(
    pl.kernel,
    mesh=_mesh,
    out_type=jax.ShapeDtypeStruct((_N, 8, 128), jnp.float32),
    scratch_types=[
        pltpu.VMEM((_IPW,), jnp.int32),
        [pltpu.VMEM((_L, 8, 128), jnp.float32)] * 2,
        [pltpu.SemaphoreType.DMA] * 2,
        [pltpu.SemaphoreType.DMA] * 2,
    ],
    compiler_params=pltpu.CompilerParams(use_tc_tiling_on_sc=False),
)
def _gather_rows(idx_hbm, table_hbm, out_hbm, idx_v, bufs, semg, semw):
    wid = lax.axis_index("s") * _NC + lax.axis_index("c")
    b0 = wid * _ROWS_PW
    pltpu.sync_copy(idx_hbm.at[pl.ds(wid * _IPW, _IPW)], idx_v)

    def gather(b, buf, sem):
        return pltpu.make_async_copy(
            table_hbm.at[idx_v.at[pl.ds(b * _LPAD, _L)]], buf, sem)

    def write(b, buf, sem):
        return pltpu.make_async_copy(
            buf, out_hbm.at[pl.ds((b0 + b) * _L, _L)], sem)

    gather(0, bufs[0], semg[0]).start()

    def body(b, carry):
        for p in (0, 1):
            gather(b + p, bufs[p], semg[p]).wait()

            @pl.when(b + p >= 1)
            def _():
                write(b + p - 1, bufs[1 - p], semw[1 - p]).wait()

            @pl.when(b + p + 1 < _ROWS_PW)
            def _():
                gather(b + p + 1, bufs[1 - p], semg[1 - p]).start()

            write(b + p, bufs[p], semw[p]).start()
        return carry

    lax.fori_loop(0, _ROWS_PW // 2, lambda i, c: body(i * 2, c), 0)
    write(_ROWS_PW - 1, bufs[1], semw[1]).wait()


_BB = 4


def _untile_body(x_ref, o_ref):
    x = x_ref[...]
    y = x.reshape(_BB * _L, _VPAD)
    y = y.reshape(_BB, _L, _VPAD)
    o_ref[...] = y[:, :, :_VOCAB]


_untile = pl.pallas_call(
    _untile_body,
    grid=(_B // _BB,),
    in_specs=[pl.BlockSpec((_BB * _L, 8, 128), lambda b: (b, 0, 0))],
    out_specs=pl.BlockSpec((_BB, _L, _VOCAB), lambda b: (b, 0, 0)),
    out_shape=jax.ShapeDtypeStruct((_B, _L, _VOCAB), jnp.float32),
)


def kernel(idx, logits):
    idx_p = jnp.pad(idx.astype(jnp.int32), ((0, 0), (0, _LPAD - _L)))
    table = jnp.pad(logits, ((0, 0), (0, _VPAD - _VOCAB)))
    rows = _gather_rows(idx_p.reshape(_B * _LPAD), table.reshape(_VOCAB, 8, 128))
    return _untile(rows)


# speedup vs baseline: 1.7005x; 1.0002x over previous
import functools

import jax
import jax.numpy as jnp
from jax import lax
from jax.experimental import pallas as pl
from jax.experimental.pallas import tpu as pltpu
from jax.experimental.pallas import tpu_sc as plsc

_VOCAB = 1000
_VPAD = 1024
_B, _L = 1024, 50
_N = _B * _L

_info = plsc.get_sparse_core_info()
_NC = _info.num_cores
_NS = _info.num_subcores
_NW = _NC * _NS
_ROWS_PW = _B // _NW
_LPAD = 56
_IPW = _ROWS_PW * _LPAD

_mesh = plsc.VectorSubcoreMesh(core_axis_name="c", subcore_axis_name="s")


@functools.partial(
    pl.kernel,
    mesh=_mesh,
    out_type=jax.ShapeDtypeStruct((_N, 8, 128), jnp.float32),
    scratch_types=[
        pltpu.VMEM((_IPW,), jnp.int32),
        [pltpu.VMEM((_L, 8, 128), jnp.float32)] * 2,
        [pltpu.SemaphoreType.DMA] * 2,
        [pltpu.SemaphoreType.DMA] * 2,
    ],
)
def _gather_rows(idx_hbm, table_hbm, out_hbm, idx_v, bufs, semg, semw):
    wid = lax.axis_index("s") * _NC + lax.axis_index("c")
    b0 = wid * _ROWS_PW
    pltpu.sync_copy(idx_hbm.at[pl.ds(wid * _IPW, _IPW)], idx_v)

    def gather(b, buf, sem):
        return pltpu.make_async_copy(
            table_hbm.at[idx_v.at[pl.ds(b * _LPAD, _L)]], buf, sem)

    def write(b, buf, sem):
        return pltpu.make_async_copy(
            buf, out_hbm.at[pl.ds((b0 + b) * _L, _L)], sem)

    gather(0, bufs[0], semg[0]).start()

    def body(b, carry):
        for p in (0, 1):
            gather(b + p, bufs[p], semg[p]).wait()

            @pl.when(b + p >= 1)
            def _():
                write(b + p - 1, bufs[1 - p], semw[1 - p]).wait()

            @pl.when(b + p + 1 < _ROWS_PW)
            def _():
                gather(b + p + 1, bufs[1 - p], semg[1 - p]).start()

            write(b + p, bufs[p], semw[p]).start()
        return carry

    lax.fori_loop(0, _ROWS_PW // 2, lambda i, c: body(i * 2, c), 0)
    write(_ROWS_PW - 1, bufs[1], semw[1]).wait()


_BB = 4


def _untile_body(x_ref, o_ref):
    x = x_ref[...]
    y = x.reshape(_BB * _L, _VPAD)
    y = y.reshape(_BB, _L, _VPAD)
    o_ref[...] = y[:, :, :_VOCAB]


_untile = pl.pallas_call(
    _untile_body,
    grid=(_B // _BB,),
    in_specs=[pl.BlockSpec((_BB * _L, 8, 128), lambda b: (b, 0, 0))],
    out_specs=pl.BlockSpec((_BB, _L, _VOCAB), lambda b: (b, 0, 0)),
    out_shape=jax.ShapeDtypeStruct((_B, _L, _VOCAB), jnp.float32),
)


def kernel(idx, logits):
    idx_p = jnp.pad(idx.astype(jnp.int32), ((0, 0), (0, _LPAD - _L)))
    table = jnp.pad(logits, ((0, 0), (0, _VPAD - _VOCAB)))
    rows = _gather_rows(idx_p.reshape(_B * _LPAD), table.reshape(_VOCAB, 8, 128))
    return _untile(rows)


# speedup vs baseline: 3.1335x; 1.8427x over previous
import functools

import jax
import jax.numpy as jnp
from jax import lax
from jax.experimental import pallas as pl
from jax.experimental.pallas import tpu as pltpu
from jax.experimental.pallas import tpu_sc as plsc

_VOCAB = 1000
_VPAD = 1024
_B, _L = 1024, 50
_N = _B * _L

_info = plsc.get_sparse_core_info()
_NC = _info.num_cores
_NS = _info.num_subcores
_NW = _NC * _NS
_BPW = _B // _NW
_LPAD = 56
_IPW = _BPW * _LPAD
_CH = 16
_NH = _BPW // _CH

_mesh = plsc.VectorSubcoreMesh(core_axis_name="c", subcore_axis_name="s")


@functools.partial(
    pl.kernel,
    mesh=_mesh,
    out_type=jax.ShapeDtypeStruct((_L * _B, 8, 128), jnp.float32),
    scratch_types=[
        pltpu.VMEM((_IPW,), jnp.int32),
        [pltpu.VMEM((_CH, 8, 128), jnp.float32)] * 2,
        [pltpu.SemaphoreType.DMA] * 2,
        [pltpu.SemaphoreType.DMA] * 2,
    ],
    compiler_params=pltpu.CompilerParams(needs_layout_passes=False),
)
def _gather_rows(idx_hbm, table_hbm, out_hbm, idx_v, bufs, semg, semw):
    wid = lax.axis_index("s") * _NC + lax.axis_index("c")
    b0 = wid * _BPW
    pltpu.sync_copy(idx_hbm.at[pl.ds(wid * _IPW, _IPW)], idx_v)
    lanes = lax.iota(jnp.int32, 16)

    def chunk_idx(c):
        l = c // _NH
        h = c % _NH
        offs = (h * _CH + lanes) * _LPAD + l
        return l, h, plsc.load_gather(idx_v, [offs])

    def gather(c, buf, sem):
        _, _, ivec = chunk_idx(c)
        return pltpu.make_async_copy(table_hbm.at[ivec], buf, sem)

    def write(c, buf, sem):
        l = c // _NH
        h = c % _NH
        row = l * _B + b0 + h * _CH
        return pltpu.make_async_copy(buf, out_hbm.at[pl.ds(row, _CH)], sem)

    gather(0, bufs[0], semg[0]).start()
    nchunks = _L * _NH

    def body(c, carry):
        for p in (0, 1):
            gather(c + p, bufs[p], semg[p]).wait()

            @pl.when(c + p >= 1)
            def _():
                write(c + p - 1, bufs[1 - p], semw[1 - p]).wait()

            @pl.when(c + p + 1 < nchunks)
            def _():
                gather(c + p + 1, bufs[1 - p], semg[1 - p]).start()

            write(c + p, bufs[p], semw[p]).start()
        return carry

    lax.fori_loop(0, nchunks // 2, lambda i, c: body(i * 2, c), 0)
    write(nchunks - 1, bufs[1], semw[1]).wait()


def _transpose_body(x_ref, o_ref):
    x = x_ref[...]
    y = x.reshape(_B, _VPAD)
    z = y.T
    o_ref[...] = z[jnp.newaxis, :_VOCAB, :]


_transpose = pl.pallas_call(
    _transpose_body,
    grid=(_L,),
    in_specs=[pl.BlockSpec((_B, 8, 128), lambda l: (l, 0, 0))],
    out_specs=pl.BlockSpec((1, _VOCAB, _B), lambda l: (l, 0, 0)),
    out_shape=jax.ShapeDtypeStruct((_L, _VOCAB, _B), jnp.float32),
)


def kernel(idx, logits):
    idx_p = jnp.pad(idx.astype(jnp.int32), ((0, 0), (0, _LPAD - _L)))
    table = jnp.pad(logits, ((0, 0), (0, _VPAD - _VOCAB)))
    rows = _gather_rows(idx_p.reshape(_B * _LPAD), table.reshape(_VOCAB, 8, 128))
    out_t = _transpose(rows)
    return jnp.transpose(out_t, (2, 0, 1))


# speedup vs baseline: 3.4270x; 1.0937x over previous
import functools

import jax
import jax.numpy as jnp
from jax import lax
from jax.experimental import pallas as pl
from jax.experimental.pallas import tpu as pltpu
from jax.experimental.pallas import tpu_sc as plsc

_VOCAB = 1000
_VPAD = 1024
_B, _L = 1024, 50
_N = _B * _L

_info = plsc.get_sparse_core_info()
_NC = _info.num_cores
_NS = _info.num_subcores
_NW = _NC * _NS
_BPW = _B // _NW
_LPAD = 56
_IPW = _BPW * _LPAD
_CH = 16
_NH = _BPW // _CH

_mesh = plsc.VectorSubcoreMesh(core_axis_name="c", subcore_axis_name="s")


@functools.partial(
    pl.kernel,
    mesh=_mesh,
    out_type=jax.ShapeDtypeStruct((_L * _B, 8, 128), jnp.float32),
    scratch_types=[
        pltpu.VMEM((_IPW,), jnp.int32),
        [pltpu.VMEM((_BPW, 8, 128), jnp.float32)] * 2,
        [pltpu.VMEM((_BPW,), jnp.int32)] * 2,
        [pltpu.SemaphoreType.DMA] * 2,
        [pltpu.SemaphoreType.DMA] * 2,
    ],
    compiler_params=pltpu.CompilerParams(needs_layout_passes=False),
)
def _gather_rows(idx_hbm, table_hbm, out_hbm, idx_v, bufs, ilst, semg, semw):
    wid = lax.axis_index("s") * _NC + lax.axis_index("c")
    b0 = wid * _BPW
    pltpu.sync_copy(idx_hbm.at[pl.ds(wid * _IPW, _IPW)], idx_v)
    lanes = lax.iota(jnp.int32, 16)

    def fill_idx(l, il):
        for h in (0, 1):
            offs = (h * 16 + lanes) * _LPAD + l
            il[pl.ds(h * 16, 16)] = plsc.load_gather(idx_v, [offs])

    def gather(il, buf, sem):
        return pltpu.make_async_copy(table_hbm.at[il], buf, sem)

    def write(l, buf, sem):
        return pltpu.make_async_copy(
            buf, out_hbm.at[pl.ds(l * _B + b0, _BPW)], sem)

    fill_idx(0, ilst[0])
    gather(ilst[0], bufs[0], semg[0]).start()

    def body(l, carry):
        for p in (0, 1):
            gather(ilst[p], bufs[p], semg[p]).wait()

            @pl.when(l + p >= 1)
            def _():
                write(l + p - 1, bufs[1 - p], semw[1 - p]).wait()

            @pl.when(l + p + 1 < _L)
            def _():
                fill_idx(l + p + 1, ilst[1 - p])
                gather(ilst[1 - p], bufs[1 - p], semg[1 - p]).start()

            write(l + p, bufs[p], semw[p]).start()
        return carry

    lax.fori_loop(0, _L // 2, lambda i, c: body(i * 2, c), 0)
    write(_L - 1, bufs[1], semw[1]).wait()


def _transpose_body(x_ref, o_ref):
    x = x_ref[...]
    y = x.reshape(_B, _VPAD)
    z = y.T
    o_ref[...] = z[jnp.newaxis, :_VOCAB, :]


_transpose = pl.pallas_call(
    _transpose_body,
    grid=(_L,),
    in_specs=[pl.BlockSpec((_B, 8, 128), lambda l: (l, 0, 0))],
    out_specs=pl.BlockSpec((1, _VOCAB, _B), lambda l: (l, 0, 0)),
    out_shape=jax.ShapeDtypeStruct((_L, _VOCAB, _B), jnp.float32),
)


def kernel(idx, logits):
    idx_p = jnp.pad(idx.astype(jnp.int32), ((0, 0), (0, _LPAD - _L)))
    table = jnp.pad(logits, ((0, 0), (0, _VPAD - _VOCAB)))
    rows = _gather_rows(idx_p.reshape(_B * _LPAD), table.reshape(_VOCAB, 8, 128))
    out_t = _transpose(rows)
    return jnp.transpose(out_t, (2, 0, 1))
